# speedup vs baseline: 1.3508x; 1.3508x over previous
import jax
import jax.numpy as jnp
from jax import lax
from jax.experimental import pallas as pl
from jax.experimental.pallas import tpu as pltpu

_N = 8192
_F_IN = 256
_F_OUT = 256
_BM = 512


def _gcn_block_kernel(seq_ref, w_ref, bias_ref, a_ref, adj_ref, out_ref, fts_ref):
    @pl.when(pl.program_id(0) == 0)
    def _compute_fts():
        fts_ref[...] = lax.dot_general(
            seq_ref[...], w_ref[...],
            (((1,), (1,)), ((), ())),
            preferred_element_type=jnp.float32,
        )

    acc = jnp.dot(adj_ref[...], fts_ref[...], preferred_element_type=jnp.float32)
    acc = acc + bias_ref[...]
    a = a_ref[0, 0]
    out_ref[...] = jnp.where(acc >= 0, acc, a * acc)


def kernel(seq, adj, W, bias, prelu_a):
    seq2 = seq.reshape(_N, _F_IN).astype(jnp.float32)
    adj2 = adj.reshape(_N, _N).astype(jnp.float32)
    bias2 = bias.reshape(1, _F_OUT).astype(jnp.float32)
    a2 = prelu_a.reshape(1, 1).astype(jnp.float32)

    grid = (_N // _BM,)
    out = pl.pallas_call(
        _gcn_block_kernel,
        grid=grid,
        in_specs=[
            pl.BlockSpec((_N, _F_IN), lambda i: (0, 0)),
            pl.BlockSpec((_F_OUT, _F_IN), lambda i: (0, 0)),
            pl.BlockSpec((1, _F_OUT), lambda i: (0, 0)),
            pl.BlockSpec((1, 1), lambda i: (0, 0)),
            pl.BlockSpec((_BM, _N), lambda i: (i, 0)),
        ],
        out_specs=pl.BlockSpec((_BM, _F_OUT), lambda i: (i, 0)),
        out_shape=jax.ShapeDtypeStruct((_N, _F_OUT), jnp.float32),
        scratch_shapes=[pltpu.VMEM((_N, _F_OUT), jnp.float32)],
    )(seq2, W.astype(jnp.float32), bias2, a2, adj2)
    return out.reshape(1, _N, _F_OUT)


# speedup vs baseline: 1.3525x; 1.0012x over previous
import jax
import jax.numpy as jnp
from jax import lax
from jax.experimental import pallas as pl
from jax.experimental.pallas import tpu as pltpu

_N = 8192
_F_IN = 256
_F_OUT = 256
_BM = 512


def _gcn_block_kernel(seq_ref, w_ref, bias_ref, a_ref, adj_ref, out_ref, fts_ref):
    @pl.when(pl.program_id(0) == 0)
    def _compute_fts():
        fts = lax.dot_general(
            seq_ref[...], w_ref[...],
            (((1,), (1,)), ((), ())),
            preferred_element_type=jnp.float32,
        )
        fts_ref[...] = fts.astype(jnp.bfloat16)

    adj_bf = adj_ref[...].astype(jnp.bfloat16)
    acc = jnp.dot(adj_bf, fts_ref[...], preferred_element_type=jnp.float32)
    acc = acc + bias_ref[...]
    a = a_ref[0, 0]
    out_ref[...] = jnp.where(acc >= 0, acc, a * acc)


def kernel(seq, adj, W, bias, prelu_a):
    seq2 = seq.reshape(_N, _F_IN).astype(jnp.float32)
    adj2 = adj.reshape(_N, _N).astype(jnp.float32)
    bias2 = bias.reshape(1, _F_OUT).astype(jnp.float32)
    a2 = prelu_a.reshape(1, 1).astype(jnp.float32)

    grid = (_N // _BM,)
    out = pl.pallas_call(
        _gcn_block_kernel,
        grid=grid,
        in_specs=[
            pl.BlockSpec((_N, _F_IN), lambda i: (0, 0)),
            pl.BlockSpec((_F_OUT, _F_IN), lambda i: (0, 0)),
            pl.BlockSpec((1, _F_OUT), lambda i: (0, 0)),
            pl.BlockSpec((1, 1), lambda i: (0, 0)),
            pl.BlockSpec((_BM, _N), lambda i: (i, 0)),
        ],
        out_specs=pl.BlockSpec((_BM, _F_OUT), lambda i: (i, 0)),
        out_shape=jax.ShapeDtypeStruct((_N, _F_OUT), jnp.float32),
        scratch_shapes=[pltpu.VMEM((_N, _F_OUT), jnp.bfloat16)],
    )(seq2, W.astype(jnp.float32), bias2, a2, adj2)
    return out.reshape(1, _N, _F_OUT)


# speedup vs baseline: 1.3543x; 1.0013x over previous
import jax
import jax.numpy as jnp
from jax import lax
from jax.experimental import pallas as pl
from jax.experimental.pallas import tpu as pltpu

_N = 8192
_F_IN = 256
_F_OUT = 256
_BM = 512


def _gcn_block_kernel(seq_ref, w_ref, bias_ref, a_ref, adj_ref, out_ref, fts_ref):
    @pl.when(pl.program_id(0) == 0)
    def _compute_fts():
        fts_ref[...] = lax.dot_general(
            seq_ref[...], w_ref[...],
            (((1,), (1,)), ((), ())),
            preferred_element_type=jnp.float32,
        )

    acc = jnp.dot(adj_ref[...], fts_ref[...], preferred_element_type=jnp.float32)
    acc = acc + bias_ref[...]
    a = a_ref[0, 0]
    out_ref[...] = jnp.where(acc >= 0, acc, a * acc)


def kernel(seq, adj, W, bias, prelu_a):
    seq2 = seq.reshape(_N, _F_IN).astype(jnp.float32)
    adj2 = adj.reshape(_N, _N).astype(jnp.float32)
    bias2 = bias.reshape(1, _F_OUT).astype(jnp.float32)
    a2 = prelu_a.reshape(1, 1).astype(jnp.float32)

    grid = (_N // _BM,)
    out = pl.pallas_call(
        _gcn_block_kernel,
        grid=grid,
        in_specs=[
            pl.BlockSpec((_N, _F_IN), lambda i: (0, 0)),
            pl.BlockSpec((_F_OUT, _F_IN), lambda i: (0, 0)),
            pl.BlockSpec((1, _F_OUT), lambda i: (0, 0)),
            pl.BlockSpec((1, 1), lambda i: (0, 0)),
            pl.BlockSpec((_BM, _N), lambda i: (i, 0)),
        ],
        out_specs=pl.BlockSpec((_BM, _F_OUT), lambda i: (i, 0)),
        out_shape=jax.ShapeDtypeStruct((_N, _F_OUT), jnp.float32),
        scratch_shapes=[pltpu.VMEM((_N, _F_OUT), jnp.float32)],
    )(seq2, W.astype(jnp.float32), bias2, a2, adj2)
    return out.reshape(1, _N, _F_OUT)
